# Initial kernel scaffold; baseline (speedup 1.0000x reference)
import jax
import jax.numpy as jnp
from jax.experimental import pallas as pl


def kernel(input_ids, subembed_table, tf_w, tf_b):
    raise NotImplementedError("write your pallas kernel here")


# speedup vs baseline: 22.2684x; 22.2684x over previous
import functools

import jax
import jax.numpy as jnp
from jax import lax
from jax.experimental import pallas as pl
from jax.experimental.pallas import tpu as pltpu
from jax.experimental.pallas import tpu_sc as plsc

_IDXW = 128
_K = 8


def _sc_gather(table, idx_2d, n_idx, d_sub):
    info = plsc.get_sparse_core_info()
    nc = info.num_cores
    nw = nc * info.num_subcores
    rows_per_w = n_idx // (nw * _IDXW)
    outer = rows_per_w // _K
    slot_rows = _K * _IDXW
    assert n_idx % (nw * _IDXW * _K) == 0

    mesh = plsc.VectorSubcoreMesh(core_axis_name="c", subcore_axis_name="s")

    @functools.partial(
        pl.kernel,
        out_type=jax.ShapeDtypeStruct((n_idx, d_sub), jnp.float32),
        mesh=mesh,
        scratch_types=[
            pltpu.VMEM((rows_per_w, _IDXW), jnp.int32),
            pltpu.VMEM((2, slot_rows, d_sub), jnp.float32),
            pltpu.SemaphoreType.DMA,
            pltpu.SemaphoreType.DMA,
        ],
        compiler_params=pltpu.CompilerParams(use_tc_tiling_on_sc=False),
    )
    def gather_kernel(table_hbm, idx_hbm, out_hbm, idx_v, rows_v, gsem, osem):
        wid = lax.axis_index("s") * nc + lax.axis_index("c")
        base = wid * rows_per_w * _IDXW
        pltpu.sync_copy(idx_hbm.at[pl.ds(wid * rows_per_w, rows_per_w)], idx_v)

        def fire_gathers(i, s):
            for j in range(_K):
                pltpu.async_copy(
                    table_hbm.at[idx_v.at[i * _K + j]],
                    rows_v.at[s, pl.ds(j * _IDXW, _IDXW)],
                    gsem,
                )

        def drain_gathers(s):
            for _ in range(_K):
                pltpu.make_async_copy(
                    table_hbm.at[idx_v.at[0]],
                    rows_v.at[s, pl.ds(0, _IDXW)],
                    gsem,
                ).wait()

        def fire_out(i, s):
            pltpu.async_copy(
                rows_v.at[s],
                out_hbm.at[pl.ds(base + i * slot_rows, slot_rows)],
                osem,
            )

        def wait_out():
            pltpu.make_async_copy(
                rows_v.at[0], out_hbm.at[pl.ds(0, slot_rows)], osem
            ).wait()

        for i in range(2):
            fire_gathers(i, i)
            drain_gathers(i)
            fire_out(i, i)

        def body(i, carry):
            s = lax.rem(i, 2)
            wait_out()
            fire_gathers(i, s)
            drain_gathers(s)
            fire_out(i, s)
            return carry

        lax.fori_loop(2, outer, body, 0)
        wait_out()
        wait_out()

    return gather_kernel(table, idx_2d)


def _tc_project(x, w_t, b):
    n, k = x.shape
    m = w_t.shape[1]
    block = 2048
    assert n % block == 0

    def body(x_ref, w_ref, b_ref, o_ref):
        o_ref[...] = (
            jnp.dot(x_ref[...], w_ref[...], preferred_element_type=jnp.float32)
            + b_ref[...]
        )

    return pl.pallas_call(
        body,
        grid=(n // block,),
        in_specs=[
            pl.BlockSpec((block, k), lambda i: (i, 0)),
            pl.BlockSpec((k, m), lambda i: (0, 0)),
            pl.BlockSpec((1, m), lambda i: (0, 0)),
        ],
        out_specs=pl.BlockSpec((block, m), lambda i: (i, 0)),
        out_shape=jax.ShapeDtypeStruct((n, m), jnp.float32),
    )(x, w_t, b.reshape(1, m))


def kernel(input_ids, subembed_table, tf_w, tf_b):
    b, l, c = input_ids.shape
    vocab, d_sub = subembed_table.shape
    d_embed = tf_w.shape[0]
    n_idx = b * l * c

    idx_2d = input_ids.reshape(n_idx // _IDXW, _IDXW).astype(jnp.int32)
    rows = _sc_gather(subembed_table, idx_2d, n_idx, d_sub)
    feats = rows.reshape(b * l, c * d_sub)
    out = _tc_project(feats, tf_w.T, tf_b)
    return out.reshape(b, l, d_embed)


# speedup vs baseline: 29.4671x; 1.3233x over previous
import functools

import jax
import jax.numpy as jnp
from jax import lax
from jax.experimental import pallas as pl
from jax.experimental.pallas import tpu as pltpu
from jax.experimental.pallas import tpu_sc as plsc

_IDXW = 128
_BLK = 512
_SLOT_BLKS = 2


def _sc_gather(table, idx_n, n_idx, d_sub, c_dim):
    info = plsc.get_sparse_core_info()
    nc = info.num_cores
    nw = nc * info.num_subcores
    per_w = n_idx // nw
    idx_rows = per_w // _IDXW
    n_blk = per_w // _BLK
    slot_rows = _SLOT_BLKS * _BLK
    outer = per_w // slot_rows
    assert n_idx % (nw * _BLK) == 0 and per_w % slot_rows == 0
    bb_per_c = _BLK // c_dim

    mesh = plsc.VectorSubcoreMesh(core_axis_name="c", subcore_axis_name="s")

    @functools.partial(
        pl.kernel,
        out_type=jax.ShapeDtypeStruct((n_idx, d_sub), jnp.float32),
        mesh=mesh,
        scratch_types=[
            pltpu.VMEM((per_w,), jnp.int32),
            pltpu.VMEM((per_w,), jnp.int32),
            pltpu.VMEM((2, slot_rows, d_sub), jnp.float32),
            pltpu.SemaphoreType.DMA,
            pltpu.SemaphoreType.DMA,
        ],
        compiler_params=pltpu.CompilerParams(
            use_tc_tiling_on_sc=False, needs_layout_passes=False
        ),
    )
    def gather_kernel(table_hbm, idx_hbm, out_hbm, idx_v, idx_p, rows_v, gsem, osem):
        wid = lax.axis_index("s") * nc + lax.axis_index("c")
        base = wid * per_w
        pltpu.sync_copy(idx_hbm.at[pl.ds(base, per_w)], idx_v)

        lanes = lax.iota(jnp.int32, 16)
        tgt0 = lanes * c_dim

        def permute_block(kk, carry):
            for c in range(c_dim):
                for v in range(bb_per_c // 16):
                    src = lanes + (kk * _BLK + c * bb_per_c + v * 16)
                    vals = plsc.load_gather(idx_v, [src])
                    tgt = tgt0 + (kk * _BLK + (v * 16) * c_dim + c)
                    plsc.store_scatter(idx_p, [tgt], vals)
            return carry

        lax.fori_loop(0, n_blk, permute_block, 0)

        def fire_gathers(i, s):
            for j in range(slot_rows // _IDXW):
                pltpu.async_copy(
                    table_hbm.at[idx_p.at[pl.ds((i * (slot_rows // _IDXW) + j) * _IDXW, _IDXW)]],
                    rows_v.at[s, pl.ds(j * _IDXW, _IDXW)],
                    gsem,
                )

        def drain_gathers(s):
            for _ in range(slot_rows // _IDXW):
                pltpu.make_async_copy(
                    table_hbm.at[idx_p.at[pl.ds(0, _IDXW)]],
                    rows_v.at[s, pl.ds(0, _IDXW)],
                    gsem,
                ).wait()

        def fire_out(i, s):
            pltpu.async_copy(
                rows_v.at[s],
                out_hbm.at[pl.ds(base + i * slot_rows, slot_rows)],
                osem,
            )

        def wait_out():
            pltpu.make_async_copy(
                rows_v.at[0], out_hbm.at[pl.ds(0, slot_rows)], osem
            ).wait()

        for i in range(2):
            fire_gathers(i, i)
            drain_gathers(i)
            fire_out(i, i)

        def body(i, carry):
            s = lax.rem(i, 2)
            wait_out()
            fire_gathers(i, s)
            drain_gathers(s)
            fire_out(i, s)
            return carry

        lax.fori_loop(2, outer, body, 0)
        wait_out()
        wait_out()

    return gather_kernel(table, idx_n)


def _tc_project(x, w_t, b):
    n, k = x.shape
    m = w_t.shape[1]
    block = 2048
    assert n % block == 0

    def body(x_ref, w_ref, b_ref, o_ref):
        o_ref[...] = (
            jnp.dot(x_ref[...], w_ref[...], preferred_element_type=jnp.float32)
            + b_ref[...]
        )

    return pl.pallas_call(
        body,
        grid=(n // block,),
        in_specs=[
            pl.BlockSpec((block, k), lambda i: (i, 0)),
            pl.BlockSpec((k, m), lambda i: (0, 0)),
            pl.BlockSpec((1, m), lambda i: (0, 0)),
        ],
        out_specs=pl.BlockSpec((block, m), lambda i: (i, 0)),
        out_shape=jax.ShapeDtypeStruct((n, m), jnp.float32),
    )(x, w_t, b.reshape(1, m))


def kernel(input_ids, subembed_table, tf_w, tf_b):
    b, l, c = input_ids.shape
    vocab, d_sub = subembed_table.shape
    d_embed = tf_w.shape[0]
    n_idx = b * l * c
    tb = b // _IDXW

    ids = input_ids.astype(jnp.int32)
    idx_n = ids.reshape(tb, _IDXW, l, c).transpose(2, 0, 3, 1).reshape(n_idx)
    rows = _sc_gather(subembed_table, idx_n, n_idx, d_sub, c)
    feats = rows.reshape(l * b, c * d_sub)
    y = _tc_project(feats, tf_w.T, tf_b)
    return y.reshape(l, b, d_embed).transpose(1, 0, 2)


# speedup vs baseline: 30.9439x; 1.0501x over previous
import functools

import jax
import jax.numpy as jnp
from jax import lax
from jax.experimental import pallas as pl
from jax.experimental.pallas import tpu as pltpu
from jax.experimental.pallas import tpu_sc as plsc

_IDXW = 128
_BLK = 512
_SLOT_IDS = 1280
_STAGE = 2560


def _sc_gather(table, idx_n, n_idx, d_sub, c_dim):
    info = plsc.get_sparse_core_info()
    nc = info.num_cores
    nw = nc * info.num_subcores
    per_w = n_idx // nw
    n_stage = per_w // _STAGE
    blk_per_stage = _STAGE // _BLK
    slot_ch = _SLOT_IDS // _IDXW
    outer = per_w // _SLOT_IDS
    assert n_idx % (nw * _BLK) == 0 and per_w % _SLOT_IDS == 0 and outer % 2 == 0
    bb_per_c = _BLK // c_dim

    mesh = plsc.VectorSubcoreMesh(core_axis_name="c", subcore_axis_name="s")

    @functools.partial(
        pl.kernel,
        out_type=jax.ShapeDtypeStruct((n_idx, d_sub), jnp.float32),
        mesh=mesh,
        scratch_types=[
            pltpu.VMEM((_STAGE,), jnp.int32),
            pltpu.VMEM((per_w,), jnp.int32),
            pltpu.VMEM((2, _SLOT_IDS, d_sub), jnp.float32),
            pltpu.SemaphoreType.DMA,
            pltpu.SemaphoreType.DMA,
            pltpu.SemaphoreType.DMA,
            pltpu.SemaphoreType.DMA,
        ],
        compiler_params=pltpu.CompilerParams(
            use_tc_tiling_on_sc=False, needs_layout_passes=False
        ),
    )
    def gather_kernel(
        table_hbm, idx_hbm, out_hbm, idx_v, idx_p, rows_v, g0, g1, o0, o1
    ):
        wid = lax.axis_index("s") * nc + lax.axis_index("c")
        base = wid * per_w

        lanes = lax.iota(jnp.int32, 16)
        tgt0 = lanes * c_dim

        def stage_chunk(ch, carry):
            pltpu.sync_copy(idx_hbm.at[pl.ds(base + ch * _STAGE, _STAGE)], idx_v)
            for kk in range(blk_per_stage):
                for c in range(c_dim):
                    for v in range(bb_per_c // 16):
                        src = lanes + (kk * _BLK + c * bb_per_c + v * 16)
                        vals = plsc.load_gather(idx_v, [src])
                        tgt = tgt0 + (
                            ch * _STAGE + kk * _BLK + (v * 16) * c_dim + c
                        )
                        plsc.store_scatter(idx_p, [tgt], vals)
            return carry

        lax.fori_loop(0, n_stage, stage_chunk, 0)

        def fire_gathers(i, s, gsem):
            for j in range(slot_ch):
                pltpu.async_copy(
                    table_hbm.at[idx_p.at[pl.ds((i * slot_ch + j) * _IDXW, _IDXW)]],
                    rows_v.at[s, pl.ds(j * _IDXW, _IDXW)],
                    gsem,
                )

        def drain_gathers(s, gsem):
            for _ in range(slot_ch):
                pltpu.make_async_copy(
                    table_hbm.at[idx_p.at[pl.ds(0, _IDXW)]],
                    rows_v.at[s, pl.ds(0, _IDXW)],
                    gsem,
                ).wait()

        def fire_out(i, s, osem):
            pltpu.async_copy(
                rows_v.at[s],
                out_hbm.at[pl.ds(base + i * _SLOT_IDS, _SLOT_IDS)],
                osem,
            )

        def wait_out(osem):
            pltpu.make_async_copy(
                rows_v.at[0], out_hbm.at[pl.ds(0, _SLOT_IDS)], osem
            ).wait()

        def retire_refill(i, s, gsem, osem, refill):
            drain_gathers(s, gsem)
            fire_out(i, s, osem)
            if refill:
                wait_out(osem)
                fire_gathers(i + 2, s, gsem)

        fire_gathers(0, 0, g0)
        fire_gathers(1, 1, g1)

        def pair_body(j, carry):
            i = j * 2
            retire_refill(i, 0, g0, o0, True)
            retire_refill(i + 1, 1, g1, o1, True)
            return carry

        lax.fori_loop(0, outer // 2 - 1, pair_body, 0)
        retire_refill(outer - 2, 0, g0, o0, False)
        retire_refill(outer - 1, 1, g1, o1, False)
        wait_out(o0)
        wait_out(o1)

    return gather_kernel(table, idx_n)


def _tc_project(x, w_t, b):
    n, k = x.shape
    m = w_t.shape[1]
    block = 4096
    assert n % block == 0

    def body(x_ref, w_ref, b_ref, o_ref):
        o_ref[...] = (
            jnp.dot(x_ref[...], w_ref[...], preferred_element_type=jnp.float32)
            + b_ref[...]
        )

    return pl.pallas_call(
        body,
        grid=(n // block,),
        in_specs=[
            pl.BlockSpec((block, k), lambda i: (i, 0)),
            pl.BlockSpec((k, m), lambda i: (0, 0)),
            pl.BlockSpec((1, m), lambda i: (0, 0)),
        ],
        out_specs=pl.BlockSpec((block, m), lambda i: (i, 0)),
        out_shape=jax.ShapeDtypeStruct((n, m), jnp.float32),
    )(x, w_t, b.reshape(1, m))


def kernel(input_ids, subembed_table, tf_w, tf_b):
    b, l, c = input_ids.shape
    vocab, d_sub = subembed_table.shape
    d_embed = tf_w.shape[0]
    n_idx = b * l * c
    tb = b // _IDXW

    ids = input_ids.astype(jnp.int32)
    idx_n = ids.reshape(tb, _IDXW, l, c).transpose(2, 0, 3, 1).reshape(n_idx)
    rows = _sc_gather(subembed_table, idx_n, n_idx, d_sub, c)
    feats = rows.reshape(l * b, c * d_sub)
    y = _tc_project(feats, tf_w.T, tf_b)
    return y.reshape(l, b, d_embed).transpose(1, 0, 2)
